# baseline (speedup 1.0000x reference)
import functools

import jax
import jax.numpy as jnp
from jax import lax
from jax.experimental import pallas as pl
from jax.experimental.pallas import tpu as pltpu
from jax.experimental.pallas import tpu_sc as plsc

_NC = 2
_NS = 16
_NW = _NC * _NS
_L = 16

_VOCAB = 1000000
_EMBD = 64
_BATCH = 4096
_SEQ = 200

_B = _BATCH * _SEQ
_CHUNK = 128
_ROWS_PER_W = _B // _NW
_CHUNKS_PER_W = _ROWS_PER_W // _CHUNK


def _emb_kernel(table_hbm, idx_hbm, emb_hbm, mask_hbm,
                idx_v, mask_v, rows_a, rows_b, sem_a, sem_b):
  wid = lax.axis_index("s") * _NC + lax.axis_index("c")
  crow = wid * _CHUNKS_PER_W

  pltpu.sync_copy(idx_hbm.at[pl.ds(crow, _CHUNKS_PER_W)], idx_v)

  rows = (rows_a, rows_b)
  sems = (sem_a, sem_b)

  def start(j, slot):
    pltpu.async_copy(table_hbm.at[idx_v.at[j]], rows[slot], sems[slot])

  def finish(j, slot):
    pltpu.make_async_copy(table_hbm.at[idx_v.at[j]], rows[slot],
                          sems[slot]).wait()
    base = wid * _ROWS_PER_W + j * _CHUNK
    pltpu.sync_copy(rows[slot], emb_hbm.at[pl.ds(base, _CHUNK)])

  start(0, 0)

  def body(jj, _):
    for b in range(2):
      j = jj * 2 + b

      @pl.when(j + 1 < _CHUNKS_PER_W)
      def _():
        start(j + 1, 1 - b)

      for i in range(_CHUNK // _L):
        v = idx_v[j, pl.ds(i * _L, _L)]
        mask_v[j, pl.ds(i * _L, _L)] = jnp.minimum(v, 1).astype(jnp.float32)

      finish(j, b)
    return 0

  lax.fori_loop(0, _CHUNKS_PER_W // 2, body, 0)

  pltpu.sync_copy(mask_v, mask_hbm.at[pl.ds(crow, _CHUNKS_PER_W)])


@jax.jit
def kernel(x, table):
  idx = x.reshape(_B // _CHUNK, _CHUNK).astype(jnp.int32)
  mesh = plsc.VectorSubcoreMesh(core_axis_name="c", subcore_axis_name="s")
  emb, mask = pl.kernel(
      _emb_kernel,
      out_type=(
          jax.ShapeDtypeStruct((_B, _EMBD), jnp.float32),
          jax.ShapeDtypeStruct((_B // _CHUNK, _CHUNK), jnp.float32),
      ),
      mesh=mesh,
      scratch_types=(
          pltpu.VMEM((_CHUNKS_PER_W, _CHUNK), jnp.int32),
          pltpu.VMEM((_CHUNKS_PER_W, _CHUNK), jnp.float32),
          pltpu.VMEM((_CHUNK, _EMBD), jnp.float32),
          pltpu.VMEM((_CHUNK, _EMBD), jnp.float32),
          pltpu.SemaphoreType.DMA,
          pltpu.SemaphoreType.DMA,
      ),
      compiler_params=pltpu.CompilerParams(use_tc_tiling_on_sc=False),
  )(table, idx)
  return emb.reshape(_BATCH, _SEQ, _EMBD), mask.reshape(_BATCH, _SEQ)


# speedup vs baseline: 1.0203x; 1.0203x over previous
import jax
import jax.numpy as jnp
from jax import lax
from jax.experimental import pallas as pl
from jax.experimental.pallas import tpu as pltpu
from jax.experimental.pallas import tpu_sc as plsc

_NC = 2
_NS = 16
_NW = _NC * _NS
_L = 16

_VOCAB = 1000000
_EMBD = 64
_BATCH = 4096
_SEQ = 200

_B = _BATCH * _SEQ
_CHUNK = 128
_ROWS_PER_W = _B // _NW
_CHUNKS_PER_W = _ROWS_PER_W // _CHUNK
_NBUF = 4


def _emb_kernel(table_hbm, idx_hbm, emb_hbm, mask_hbm,
                idx_v, mask_v, rows, gsems, psems):
  wid = lax.axis_index("s") * _NC + lax.axis_index("c")
  crow = wid * _CHUNKS_PER_W

  pltpu.sync_copy(idx_hbm.at[pl.ds(crow, _CHUNKS_PER_W)], idx_v)

  def start_gather(j, b):
    pltpu.async_copy(table_hbm.at[idx_v.at[j]], rows[b], gsems[b])

  def wait_gather(j, b):
    pltpu.make_async_copy(table_hbm.at[idx_v.at[j]], rows[b],
                          gsems[b]).wait()

  def out_slice(j):
    return emb_hbm.at[pl.ds(wid * _ROWS_PER_W + j * _CHUNK, _CHUNK)]

  def start_put(j, b):
    pltpu.async_copy(rows[b], out_slice(j), psems[b])

  def wait_put(j, b):
    pltpu.make_async_copy(rows[b], out_slice(j), psems[b]).wait()

  def mask(j):
    for i in range(_CHUNK // _L):
      v = idx_v[j, pl.ds(i * _L, _L)]
      mask_v[j, pl.ds(i * _L, _L)] = jnp.minimum(v, 1).astype(jnp.float32)

  def step(j, b, *, head, tail):
    if not head:
      wait_put(j - 2, (b + 2) % _NBUF)
    if not tail:
      start_gather(j + 2, (b + 2) % _NBUF)
    wait_gather(j, b)
    start_put(j, b)
    mask(j)

  start_gather(0, 0)
  start_gather(1, 1)
  for b in range(_NBUF):
    step(b, b, head=(b < 2), tail=False)

  def body(jj, _):
    for b in range(_NBUF):
      step(jj * _NBUF + b, b, head=False, tail=False)
    return 0

  lax.fori_loop(1, _CHUNKS_PER_W // _NBUF - 1, body, 0)

  last = _CHUNKS_PER_W - _NBUF
  for b in range(_NBUF):
    step(last + b, b, head=False, tail=(b >= 2))
  wait_put(_CHUNKS_PER_W - 2, (_CHUNKS_PER_W - 2) % _NBUF)
  wait_put(_CHUNKS_PER_W - 1, (_CHUNKS_PER_W - 1) % _NBUF)

  pltpu.sync_copy(mask_v, mask_hbm.at[pl.ds(crow, _CHUNKS_PER_W)])


@jax.jit
def kernel(x, table):
  idx = x.reshape(_B // _CHUNK, _CHUNK).astype(jnp.int32)
  mesh = plsc.VectorSubcoreMesh(core_axis_name="c", subcore_axis_name="s")
  emb, mask = pl.kernel(
      _emb_kernel,
      out_type=(
          jax.ShapeDtypeStruct((_B, _EMBD), jnp.float32),
          jax.ShapeDtypeStruct((_B // _CHUNK, _CHUNK), jnp.float32),
      ),
      mesh=mesh,
      scratch_types=(
          pltpu.VMEM((_CHUNKS_PER_W, _CHUNK), jnp.int32),
          pltpu.VMEM((_CHUNKS_PER_W, _CHUNK), jnp.float32),
          tuple(pltpu.VMEM((_CHUNK, _EMBD), jnp.float32)
                for _ in range(_NBUF)),
          tuple(pltpu.SemaphoreType.DMA for _ in range(_NBUF)),
          tuple(pltpu.SemaphoreType.DMA for _ in range(_NBUF)),
      ),
      compiler_params=pltpu.CompilerParams(use_tc_tiling_on_sc=False),
  )(table, idx)
  return emb.reshape(_BATCH, _SEQ, _EMBD), mask.reshape(_BATCH, _SEQ)
